# baseline (speedup 1.0000x reference)
import functools

import jax
import jax.numpy as jnp
from jax import lax
from jax.experimental import pallas as pl
from jax.experimental.pallas import tpu as pltpu
from jax.experimental.pallas import tpu_sc as plsc

N_NODES = 50000
N_REL = 16
HID = 64
N_EDGES = 800000
BATCH = 4096

NC = 2
NS = 16
NW = NC * NS
L = 16

HALF = 25600
NPAD = NC * HALF
HROWS = HALF // 2
NPADR = NPAD * N_REL

CH = 128
TOTC = N_EDGES // CH

ZW = 6400
CPS = NPADR // NS
WPW = NPADR // NW
ZR = 80
RPS = HROWS // NS

F32 = jnp.float32
I32 = jnp.int32

_sc_mesh = plsc.VectorSubcoreMesh(
    core_axis_name="c", subcore_axis_name="s", num_cores=NC, num_subcores=NS)
_sc_params = pltpu.CompilerParams(needs_layout_passes=False)
_sc_params_lin = pltpu.CompilerParams(
    needs_layout_passes=False, use_tc_tiling_on_sc=False)


def _k1_body(dst_hbm, et_hbm, cnt_hbm, dst_v, et_v, key_v, ones_v, z_v,
             cnt_sh):
    c = lax.axis_index("c")
    s = lax.axis_index("s")
    w = c * NS + s
    zeros = jnp.zeros((L,), F32)
    ones = jnp.ones((L,), F32)

    def zf(i, _):
        z_v[pl.ds(i * L, L)] = zeros
        return 0

    lax.fori_loop(0, ZW // L, zf, 0)
    for i in range(8):
        ones_v[pl.ds(i * L, L)] = ones
    for i in range(CPS // ZW):
        pltpu.sync_copy(z_v, cnt_sh.at[pl.ds(s * CPS + i * ZW, ZW)])
    plsc.subcore_barrier()

    nch = (TOTC // NW) + jnp.where(w < TOTC % NW, 1, 0)

    def chunk(g, _):
        off = (g * NW + w) * CH
        pltpu.sync_copy(dst_hbm.at[pl.ds(off, CH)], dst_v)
        pltpu.sync_copy(et_hbm.at[pl.ds(off, CH)], et_v)

        def grp(j, _):
            sl = pl.ds(j * L, L)
            key_v[sl] = dst_v[sl] * N_REL + et_v[sl]
            return 0

        lax.fori_loop(0, CH // L, grp, 0)
        pltpu.sync_copy(ones_v, cnt_sh.at[key_v], add=True)
        return 0

    lax.fori_loop(0, nch, chunk, 0)
    plsc.subcore_barrier()
    for i in range(CPS // ZW):
        pltpu.sync_copy(cnt_sh.at[pl.ds(s * CPS + i * ZW, ZW)],
                        cnt_hbm.at[pl.ds(c * NPADR + s * CPS + i * ZW, ZW)])


_k1 = pl.kernel(
    _k1_body,
    compiler_params=_sc_params,
    out_type=jax.ShapeDtypeStruct((NC * NPADR,), F32),
    mesh=_sc_mesh,
    scratch_types=[
        pltpu.VMEM((CH,), I32),
        pltpu.VMEM((CH,), I32),
        pltpu.VMEM((CH,), I32),
        pltpu.VMEM((CH,), F32),
        pltpu.VMEM((ZW,), F32),
        pltpu.VMEM_SHARED((NPADR,), F32),
    ],
)


def _k1b_body(cnt_hbm, recip_hbm, a_v, b_v):
    c = lax.axis_index("c")
    s = lax.axis_index("s")
    base = (c * NS + s) * WPW

    def blk(i, _):
        off = base + i * ZW
        pltpu.sync_copy(cnt_hbm.at[pl.ds(off, ZW)], a_v)
        pltpu.sync_copy(cnt_hbm.at[pl.ds(NPADR + off, ZW)], b_v)

        def grp(j, _):
            sl = pl.ds(j * L, L)
            a_v[sl] = 1.0 / jnp.maximum(a_v[sl] + b_v[sl], 1.0)
            return 0

        lax.fori_loop(0, ZW // L, grp, 0)
        pltpu.sync_copy(a_v, recip_hbm.at[pl.ds(off, ZW)])
        return 0

    lax.fori_loop(0, WPW // ZW, blk, 0)


_k1b = pl.kernel(
    _k1b_body,
    compiler_params=_sc_params,
    out_type=jax.ShapeDtypeStruct((NPADR,), F32),
    mesh=_sc_mesh,
    scratch_types=[
        pltpu.VMEM((ZW,), F32),
        pltpu.VMEM((ZW,), F32),
    ],
)


BSTR = ((TOTC + NW - 1) // NW) * CH


def _k2_body(src_hbm, dst_hbm, et_hbm, g_out, k_out, l_out, cnt_out,
             sr_v, ds_v, tp_v, gq_v, kq_v, lq_v, dq_v):
    c = lax.axis_index("c")
    s = lax.axis_index("s")
    w = c * NS + s
    ones_i = jnp.ones((L,), I32)
    iota = plsc.cumsum(ones_i) - 1
    base0 = w * BSTR
    base1 = (NW + w) * BSTR
    nch = (TOTC // NW) + jnp.where(w < TOTC % NW, 1, 0)

    def chunk(g, carry):
        off = (g * NW + w) * CH
        pltpu.sync_copy(src_hbm.at[pl.ds(off, CH)], sr_v)
        pltpu.sync_copy(dst_hbm.at[pl.ds(off, CH)], ds_v)
        pltpu.sync_copy(et_hbm.at[pl.ds(off, CH)], tp_v)

        def grp(j, carry2):
            o0, o1 = carry2
            sl = pl.ds(j * L, L)
            sv = sr_v[sl]
            dvv = ds_v[sl]
            tv = tp_v[sl]
            gq_v[sl] = (lax.shift_right_logical(tv, 1) * (2 * N_NODES)
                        + sv * 2 + (tv & 1))
            kq_v[sl] = dvv * N_REL + tv
            own0 = dvv < HALF
            lq_v[sl] = jnp.where(own0, dvv, dvv - HALF)
            m = jnp.where(own0, ones_i, 0)
            ps = plsc.cumsum(m)
            r0 = ps - m
            r1 = iota - r0
            dq_v[sl] = jnp.where(own0, base0 + o0 + r0, base1 + o1 + r1)
            t0 = ps[L - 1]
            return (o0 + t0, o1 + (L - t0))

        carry = lax.fori_loop(0, CH // L, grp, carry)
        pltpu.sync_copy(gq_v, g_out.at[dq_v])
        pltpu.sync_copy(kq_v, k_out.at[dq_v])
        pltpu.sync_copy(lq_v, l_out.at[dq_v])
        return carry

    o0, o1 = lax.fori_loop(0, nch, chunk, (jnp.int32(0), jnp.int32(0)))
    cntv = jnp.where(iota == 0, o0, jnp.where(iota == 1, o1, 0))
    sr_v[pl.ds(0, L)] = cntv
    pltpu.sync_copy(sr_v.at[pl.ds(0, L)], cnt_out.at[pl.ds(w * L, L)])


_k2 = pl.kernel(
    _k2_body,
    compiler_params=_sc_params,
    out_type=[
        jax.ShapeDtypeStruct((2 * NW * BSTR,), I32),
        jax.ShapeDtypeStruct((2 * NW * BSTR,), I32),
        jax.ShapeDtypeStruct((2 * NW * BSTR,), I32),
        jax.ShapeDtypeStruct((NW * L,), I32),
    ],
    mesh=_sc_mesh,
    scratch_types=[
        pltpu.VMEM((CH,), I32),
        pltpu.VMEM((CH,), I32),
        pltpu.VMEM((CH,), I32),
        pltpu.VMEM((CH,), I32),
        pltpu.VMEM((CH,), I32),
        pltpu.VMEM((CH,), I32),
        pltpu.VMEM((CH,), I32),
    ],
)


ZR2 = 160
RPS2 = HALF // NS


def _k3_body(y_hbm, g_hbm, k_hbm, l_hbm, cnt_hbm, recip_hbm, out_hbm,
             gq_v, kq_v, lq_v, sidx_v, scl_v, smsk_v, rows_v, z2_v, cw_v,
             sem, out_sh):
    c = lax.axis_index("c")
    s = lax.axis_index("s")
    zeros = jnp.zeros((L,), F32)
    ones_i = jnp.ones((L,), I32)
    iota = plsc.cumsum(ones_i) - 1

    def zf(i, _):
        for q in range(HID // L):
            z2_v[i, pl.ds(q * L, L)] = zeros
        return 0

    lax.fori_loop(0, ZR2, zf, 0)
    for i in range(RPS2 // ZR2):
        pltpu.sync_copy(z2_v, out_sh.at[pl.ds(s * RPS2 + i * ZR2, ZR2)])
    plsc.subcore_barrier()

    for half in range(2):
        w = 2 * s + half
        pltpu.sync_copy(cnt_hbm.at[pl.ds(w * L, L)], cw_v)
        crow = cw_v[pl.ds(0, L)]
        cnt = jnp.sum(jnp.where(iota == c, crow, 0))
        nchb = lax.shift_right_logical(cnt + (CH - 1), 7)
        base = (c * NW + w) * BSTR

        def chunk(g, _):
            off = base + g * CH
            pltpu.sync_copy(g_hbm.at[pl.ds(off, CH)], gq_v)
            pltpu.sync_copy(k_hbm.at[pl.ds(off, CH)], kq_v)
            pltpu.sync_copy(l_hbm.at[pl.ds(off, CH)], lq_v)
            rem = cnt - g * CH

            def pre(j, _):
                sl = pl.ds(j * L, L)
                okv = (j * L + iota) < rem
                smsk_v[sl] = jnp.where(okv, 1.0, 0.0)
                sidx_v[sl] = jnp.clip(lq_v[sl], 0, HALF - 1)
                gq_v[sl] = jnp.clip(gq_v[sl], 0, N_REL * N_NODES - 1)
                kq_v[sl] = jnp.clip(kq_v[sl], 0, NPADR - 1)
                return 0

            lax.fori_loop(0, CH // L, pre, 0)
            cp1 = pltpu.async_copy(recip_hbm.at[kq_v], scl_v, sem)
            cp2 = pltpu.async_copy(y_hbm.at[gq_v], rows_v, sem)
            cp1.wait()
            cp2.wait()

            def pe(j, _):
                sl = pl.ds(j * L, L)
                sv16 = smsk_v[sl] * scl_v[sl]
                for t in range(L):
                    e = j * L + t
                    sv = sv16[t]
                    for q in range(HID // L):
                        rows_v[e, pl.ds(q * L, L)] = (
                            rows_v[e, pl.ds(q * L, L)] * sv)
                return 0

            lax.fori_loop(0, CH // L, pe, 0)
            pltpu.sync_copy(rows_v, out_sh.at[sidx_v], add=True)
            return 0

        lax.fori_loop(0, nchb, chunk, 0)

    plsc.subcore_barrier()
    for i in range(RPS2 // ZR2):
        pltpu.sync_copy(out_sh.at[pl.ds(s * RPS2 + i * ZR2, ZR2)],
                        out_hbm.at[pl.ds(c * HALF + s * RPS2 + i * ZR2, ZR2)])


_k3 = pl.kernel(
    _k3_body,
    compiler_params=_sc_params_lin,
    out_type=jax.ShapeDtypeStruct((NC * HALF, HID), F32),
    mesh=_sc_mesh,
    scratch_types=[
        pltpu.VMEM((CH,), I32),
        pltpu.VMEM((CH,), I32),
        pltpu.VMEM((CH,), I32),
        pltpu.VMEM((CH,), I32),
        pltpu.VMEM((CH,), F32),
        pltpu.VMEM((CH,), F32),
        pltpu.VMEM((CH, HID), F32),
        pltpu.VMEM((ZR2, HID), F32),
        pltpu.VMEM((L,), I32),
        pltpu.SemaphoreType.DMA,
        pltpu.VMEM_SHARED((HALF, HID), F32),
    ],
)


PPS = BATCH // NW


def _k4_body(x_hbm, d1_hbm, d2_hbm, out_hbm, idx_v, rows_v, sem):
    c = lax.axis_index("c")
    s = lax.axis_index("s")
    base = (c * NS + s) * PPS
    pltpu.sync_copy(d1_hbm.at[pl.ds(base, PPS)], idx_v)
    pltpu.async_copy(x_hbm.at[idx_v], rows_v, sem).wait()
    pltpu.sync_copy(rows_v, out_hbm.at[0, pl.ds(base, PPS)])
    pltpu.sync_copy(d2_hbm.at[pl.ds(base, PPS)], idx_v)
    pltpu.async_copy(x_hbm.at[idx_v], rows_v, sem).wait()
    pltpu.sync_copy(rows_v, out_hbm.at[1, pl.ds(base, PPS)])


_k4 = pl.kernel(
    _k4_body,
    compiler_params=_sc_params,
    out_type=jax.ShapeDtypeStruct((2, BATCH, 2 * HID), F32),
    mesh=_sc_mesh,
    scratch_types=[
        pltpu.VMEM((PPS,), I32),
        pltpu.VMEM((PPS, 2 * HID), F32),
        pltpu.SemaphoreType.DMA,
    ],
)


ROW_BLK = 2000
NB = N_NODES // ROW_BLK
NQ = N_REL // 2
ROW_BLK2 = 400
NB2 = N_NODES // ROW_BLK2


def _y_kernel(x_ref, wcat_ref, y_ref):
    y_ref[...] = jnp.dot(x_ref[...], wcat_ref[...],
                         preferred_element_type=F32)


def _y_matmul(x, wcat):
    return pl.pallas_call(
        _y_kernel,
        grid=(NQ, NB),
        in_specs=[
            pl.BlockSpec((ROW_BLK, HID), lambda q, i: (i, 0)),
            pl.BlockSpec((HID, 2 * HID), lambda q, i: (0, q)),
        ],
        out_specs=pl.BlockSpec((ROW_BLK, 2 * HID),
                               lambda q, i: (q * NB + i, 0)),
        out_shape=jax.ShapeDtypeStruct((NQ * N_NODES, 2 * HID), F32),
    )(x, wcat)


def _layer_kernel(relu, pad, x_ref, out2_ref, wroot_ref, b_ref, xn_ref):
    h = (jnp.dot(x_ref[...], wroot_ref[...], preferred_element_type=F32)
         + b_ref[...] + out2_ref[...])
    if relu:
        h = jnp.maximum(h, 0.0)
    if pad:
        h = jnp.concatenate([h, jnp.zeros_like(h)], axis=1)
    xn_ref[...] = h


def _layer_update(x, out2, wroot, b, relu, pad):
    n = x.shape[0]
    ow = 2 * HID if pad else HID
    return pl.pallas_call(
        functools.partial(_layer_kernel, relu, pad),
        grid=(NB2,),
        in_specs=[
            pl.BlockSpec((ROW_BLK2, HID), lambda i: (i, 0)),
            pl.BlockSpec((ROW_BLK2, HID), lambda i: (i, 0)),
            pl.BlockSpec((HID, HID), lambda i: (0, 0)),
            pl.BlockSpec((1, HID), lambda i: (0, 0)),
        ],
        out_specs=pl.BlockSpec((ROW_BLK2, ow), lambda i: (i, 0)),
        out_shape=jax.ShapeDtypeStruct((n, ow), F32),
    )(x, out2, wroot, b.reshape(1, HID))


def _head_kernel(h1_ref, h2_ref, w1a_ref, w1b_ref, b1_ref, w2_ref, b2_ref,
                 o_ref):
    h = (jnp.dot(h1_ref[...], w1a_ref[...], preferred_element_type=F32)
         + jnp.dot(h2_ref[...], w1b_ref[...], preferred_element_type=F32)
         + b1_ref[...])
    h = jnp.maximum(h, 0.0)
    o_ref[...] = jnp.dot(h, w2_ref[...], preferred_element_type=F32) + b2_ref[...]


def _head(h1, h2, wc1, bc1, wc2, bc2):
    zpad = jnp.zeros((HID, HID), F32)
    w1a = jnp.concatenate([wc1[:HID], zpad], axis=0)
    w1b = jnp.concatenate([wc1[HID:], zpad], axis=0)
    return pl.pallas_call(
        _head_kernel,
        out_shape=jax.ShapeDtypeStruct((BATCH, N_REL), F32),
    )(h1, h2, w1a, w1b, bc1.reshape(1, HID), wc2, bc2.reshape(1, N_REL))


def kernel(edge_index, edge_type, drug1_idx, drug2_idx, emb, Wr1, Wroot1, b1,
           Wr2, Wroot2, b2, Wc1, bc1, Wc2, bc2):
    src = edge_index[0].astype(I32)
    dst = edge_index[1].astype(I32)
    et = edge_type.astype(I32)
    d1 = drug1_idx.astype(I32)
    d2 = drug2_idx.astype(I32)

    cnt = _k1(dst, et)
    recip = _k1b(cnt)
    g_p, k_p, l_p, cnts = _k2(src, dst, et)

    Wcat1 = jnp.transpose(Wr1, (1, 0, 2)).reshape(HID, N_REL * HID)
    Wcat2 = jnp.transpose(Wr2, (1, 0, 2)).reshape(HID, N_REL * HID)

    y1 = _y_matmul(emb, Wcat1).reshape(2 * NQ * N_NODES, HID)
    out2_1 = _k3(y1, g_p, k_p, l_p, cnts, recip)[:N_NODES]
    x1 = _layer_update(emb, out2_1, Wroot1, b1, relu=True, pad=False)
    y2 = _y_matmul(x1, Wcat2).reshape(2 * NQ * N_NODES, HID)
    out2_2 = _k3(y2, g_p, k_p, l_p, cnts, recip)[:N_NODES]
    x2 = _layer_update(x1, out2_2, Wroot2, b2, relu=False, pad=True)

    h12 = _k4(x2, d1, d2)
    return _head(h12[0], h12[1], Wc1, bc1, Wc2, bc2)


# speedup vs baseline: 2.6610x; 2.6610x over previous
import functools

import jax
import jax.numpy as jnp
from jax import lax
from jax.experimental import pallas as pl
from jax.experimental.pallas import tpu as pltpu
from jax.experimental.pallas import tpu_sc as plsc

N_NODES = 50000
N_REL = 16
HID = 64
N_EDGES = 800000
BATCH = 4096

NC = 2
NS = 16
NW = NC * NS
L = 16

HALF = 25600
NPAD = NC * HALF
HROWS = HALF // 2
NPADR = NPAD * N_REL

CH = 128
TOTC = N_EDGES // CH

ZW = 6400
CPS = NPADR // NS
WPW = NPADR // NW
ZR = 80
RPS = HROWS // NS

F32 = jnp.float32
I32 = jnp.int32

_sc_mesh = plsc.VectorSubcoreMesh(
    core_axis_name="c", subcore_axis_name="s", num_cores=NC, num_subcores=NS)
_sc_params = pltpu.CompilerParams(needs_layout_passes=False)
_sc_params_lin = pltpu.CompilerParams(
    needs_layout_passes=False, use_tc_tiling_on_sc=False)


def _k1_body(dst_hbm, et_hbm, cnt_hbm, dst_v, et_v, key_v, ones_v, z_v,
             cnt_sh):
    c = lax.axis_index("c")
    s = lax.axis_index("s")
    w = c * NS + s
    zeros = jnp.zeros((L,), F32)
    ones = jnp.ones((L,), F32)

    def zf(i, _):
        z_v[pl.ds(i * L, L)] = zeros
        return 0

    lax.fori_loop(0, ZW // L, zf, 0)
    for i in range(8):
        ones_v[pl.ds(i * L, L)] = ones
    for i in range(CPS // ZW):
        pltpu.sync_copy(z_v, cnt_sh.at[pl.ds(s * CPS + i * ZW, ZW)])
    plsc.subcore_barrier()

    nch = (TOTC // NW) + jnp.where(w < TOTC % NW, 1, 0)

    def chunk(g, _):
        off = (g * NW + w) * CH
        pltpu.sync_copy(dst_hbm.at[pl.ds(off, CH)], dst_v)
        pltpu.sync_copy(et_hbm.at[pl.ds(off, CH)], et_v)

        def grp(j, _):
            sl = pl.ds(j * L, L)
            key_v[sl] = dst_v[sl] * N_REL + et_v[sl]
            return 0

        lax.fori_loop(0, CH // L, grp, 0)
        pltpu.sync_copy(ones_v, cnt_sh.at[key_v], add=True)
        return 0

    lax.fori_loop(0, nch, chunk, 0)
    plsc.subcore_barrier()
    for i in range(CPS // ZW):
        pltpu.sync_copy(cnt_sh.at[pl.ds(s * CPS + i * ZW, ZW)],
                        cnt_hbm.at[pl.ds(c * NPADR + s * CPS + i * ZW, ZW)])


_k1 = pl.kernel(
    _k1_body,
    compiler_params=_sc_params,
    out_type=jax.ShapeDtypeStruct((NC * NPADR,), F32),
    mesh=_sc_mesh,
    scratch_types=[
        pltpu.VMEM((CH,), I32),
        pltpu.VMEM((CH,), I32),
        pltpu.VMEM((CH,), I32),
        pltpu.VMEM((CH,), F32),
        pltpu.VMEM((ZW,), F32),
        pltpu.VMEM_SHARED((NPADR,), F32),
    ],
)


def _k1b_body(cnt_hbm, recip_hbm, a_v, b_v):
    c = lax.axis_index("c")
    s = lax.axis_index("s")
    base = (c * NS + s) * WPW

    def blk(i, _):
        off = base + i * ZW
        pltpu.sync_copy(cnt_hbm.at[pl.ds(off, ZW)], a_v)
        pltpu.sync_copy(cnt_hbm.at[pl.ds(NPADR + off, ZW)], b_v)

        def grp(j, _):
            sl = pl.ds(j * L, L)
            a_v[sl] = 1.0 / jnp.maximum(a_v[sl] + b_v[sl], 1.0)
            return 0

        lax.fori_loop(0, ZW // L, grp, 0)
        pltpu.sync_copy(a_v, recip_hbm.at[pl.ds(off, ZW)])
        return 0

    lax.fori_loop(0, WPW // ZW, blk, 0)


_k1b = pl.kernel(
    _k1b_body,
    compiler_params=_sc_params,
    out_type=jax.ShapeDtypeStruct((NPADR,), F32),
    mesh=_sc_mesh,
    scratch_types=[
        pltpu.VMEM((ZW,), F32),
        pltpu.VMEM((ZW,), F32),
    ],
)


BCAP = ((TOTC + NS - 1) // NS) * CH


def _k2_body(src_hbm, dst_hbm, et_hbm, a_out, b_out, cnt_out,
             sr_v, ds_v, tp_v, aq_v, bq_v, dq_v, a_sh, b_sh):
    c = lax.axis_index("c")
    s = lax.axis_index("s")
    ones_i = jnp.ones((L,), I32)
    iota = plsc.cumsum(ones_i) - 1
    node_base = c * HALF
    bbase = s * BCAP
    dump = NS * BCAP
    nch = (TOTC // NS) + jnp.where(s < TOTC % NS, 1, 0)

    def chunk(g, o):
        off = (g * NS + s) * CH
        pltpu.sync_copy(src_hbm.at[pl.ds(off, CH)], sr_v)
        pltpu.sync_copy(dst_hbm.at[pl.ds(off, CH)], ds_v)
        pltpu.sync_copy(et_hbm.at[pl.ds(off, CH)], tp_v)

        def grp(j, o2):
            sl = pl.ds(j * L, L)
            sv = sr_v[sl]
            dvv = ds_v[sl]
            tv = tp_v[sl]
            aq_v[sl] = (lax.shift_right_logical(tv, 1) * (2 * N_NODES)
                        + sv * 2 + (tv & 1))
            bq_v[sl] = dvv * N_REL + tv
            lvv = dvv - node_base
            own = (lvv >= 0) & (lvv < HALF)
            m = jnp.where(own, ones_i, 0)
            ps = plsc.cumsum(m)
            dq_v[sl] = jnp.where(own, bbase + o2 + ps - m, dump + iota)
            return o2 + ps[L - 1]

        o = lax.fori_loop(0, CH // L, grp, o)
        pltpu.sync_copy(aq_v, a_sh.at[dq_v])
        pltpu.sync_copy(bq_v, b_sh.at[dq_v])
        return o

    o = lax.fori_loop(0, nch, chunk, jnp.int32(0))
    cntv = jnp.where(iota == 0, o, 0)
    sr_v[pl.ds(0, L)] = cntv
    pltpu.sync_copy(sr_v.at[pl.ds(0, L)],
                    cnt_out.at[pl.ds((c * NS + s) * L, L)])
    gbase = (c * NS + s) * BCAP
    nblk = lax.shift_right_logical(o + (CH - 1), 7)

    def dr(i, _):
        pltpu.sync_copy(a_sh.at[pl.ds(bbase + i * CH, CH)],
                        a_out.at[pl.ds(gbase + i * CH, CH)])
        pltpu.sync_copy(b_sh.at[pl.ds(bbase + i * CH, CH)],
                        b_out.at[pl.ds(gbase + i * CH, CH)])
        return 0

    lax.fori_loop(0, nblk, dr, 0)


_k2 = pl.kernel(
    _k2_body,
    compiler_params=_sc_params,
    out_type=[
        jax.ShapeDtypeStruct((NC * NS * BCAP,), I32),
        jax.ShapeDtypeStruct((NC * NS * BCAP,), I32),
        jax.ShapeDtypeStruct((NC * NS * L,), I32),
    ],
    mesh=_sc_mesh,
    scratch_types=[
        pltpu.VMEM((CH,), I32),
        pltpu.VMEM((CH,), I32),
        pltpu.VMEM((CH,), I32),
        pltpu.VMEM((CH,), I32),
        pltpu.VMEM((CH,), I32),
        pltpu.VMEM((CH,), I32),
        pltpu.VMEM_SHARED((NS * BCAP + L,), I32),
        pltpu.VMEM_SHARED((NS * BCAP + L,), I32),
    ],
)


ZR2 = 160
RPS2 = HALF // NS


def _k3_body(y_hbm, a_hbm, b_hbm, cnt_hbm, recip_hbm, out_hbm,
             gq_v, kq_v, sidx_v, scl_v, smsk_v, rows_v, z2_v, cw_v,
             sem, out_sh):
    c = lax.axis_index("c")
    s = lax.axis_index("s")
    zeros = jnp.zeros((L,), F32)
    ones_i = jnp.ones((L,), I32)
    iota = plsc.cumsum(ones_i) - 1
    node_base = c * HALF

    def zf(i, _):
        for q in range(HID // L):
            z2_v[i, pl.ds(q * L, L)] = zeros
        return 0

    lax.fori_loop(0, ZR2, zf, 0)
    for i in range(RPS2 // ZR2):
        pltpu.sync_copy(z2_v, out_sh.at[pl.ds(s * RPS2 + i * ZR2, ZR2)])
    plsc.subcore_barrier()

    pltpu.sync_copy(cnt_hbm.at[pl.ds((c * NS + s) * L, L)], cw_v)
    crow = cw_v[pl.ds(0, L)]
    cnt = crow[0]
    nchb = lax.shift_right_logical(cnt + (CH - 1), 7)
    base = (c * NS + s) * BCAP

    def chunk(g, _):
        off = base + g * CH
        pltpu.sync_copy(a_hbm.at[pl.ds(off, CH)], gq_v)
        pltpu.sync_copy(b_hbm.at[pl.ds(off, CH)], kq_v)
        rem = cnt - g * CH

        def pre(j, _):
            sl = pl.ds(j * L, L)
            okv = (j * L + iota) < rem
            smsk_v[sl] = jnp.where(okv, 1.0, 0.0)
            kv = jnp.clip(kq_v[sl], 0, NPADR - 1)
            kq_v[sl] = kv
            sidx_v[sl] = jnp.clip(
                lax.shift_right_logical(kv, 4) - node_base, 0, HALF - 1)
            gq_v[sl] = jnp.clip(gq_v[sl], 0, N_REL * N_NODES - 1)
            return 0

        lax.fori_loop(0, CH // L, pre, 0)
        cp1 = pltpu.async_copy(recip_hbm.at[kq_v], scl_v, sem)
        cp2 = pltpu.async_copy(y_hbm.at[gq_v], rows_v, sem)
        cp1.wait()
        cp2.wait()

        def pe(j, _):
            sl = pl.ds(j * L, L)
            sv16 = smsk_v[sl] * scl_v[sl]
            for t in range(L):
                e = j * L + t
                sv = sv16[t]
                for q in range(HID // L):
                    rows_v[e, pl.ds(q * L, L)] = (
                        rows_v[e, pl.ds(q * L, L)] * sv)
            return 0

        lax.fori_loop(0, CH // L, pe, 0)
        pltpu.sync_copy(rows_v, out_sh.at[sidx_v], add=True)
        return 0

    lax.fori_loop(0, nchb, chunk, 0)

    plsc.subcore_barrier()
    for i in range(RPS2 // ZR2):
        pltpu.sync_copy(out_sh.at[pl.ds(s * RPS2 + i * ZR2, ZR2)],
                        out_hbm.at[pl.ds(c * HALF + s * RPS2 + i * ZR2, ZR2)])


_k3 = pl.kernel(
    _k3_body,
    compiler_params=_sc_params_lin,
    out_type=jax.ShapeDtypeStruct((NC * HALF, HID), F32),
    mesh=_sc_mesh,
    scratch_types=[
        pltpu.VMEM((CH,), I32),
        pltpu.VMEM((CH,), I32),
        pltpu.VMEM((CH,), I32),
        pltpu.VMEM((CH,), F32),
        pltpu.VMEM((CH,), F32),
        pltpu.VMEM((CH, HID), F32),
        pltpu.VMEM((ZR2, HID), F32),
        pltpu.VMEM((L,), I32),
        pltpu.SemaphoreType.DMA,
        pltpu.VMEM_SHARED((HALF, HID), F32),
    ],
)


PPS = BATCH // NW


def _k4_body(x_hbm, d1_hbm, d2_hbm, out_hbm, idx_v, rows_v, sem):
    c = lax.axis_index("c")
    s = lax.axis_index("s")
    base = (c * NS + s) * PPS
    pltpu.sync_copy(d1_hbm.at[pl.ds(base, PPS)], idx_v)
    pltpu.async_copy(x_hbm.at[idx_v], rows_v, sem).wait()
    pltpu.sync_copy(rows_v, out_hbm.at[0, pl.ds(base, PPS)])
    pltpu.sync_copy(d2_hbm.at[pl.ds(base, PPS)], idx_v)
    pltpu.async_copy(x_hbm.at[idx_v], rows_v, sem).wait()
    pltpu.sync_copy(rows_v, out_hbm.at[1, pl.ds(base, PPS)])


_k4 = pl.kernel(
    _k4_body,
    compiler_params=_sc_params,
    out_type=jax.ShapeDtypeStruct((2, BATCH, 2 * HID), F32),
    mesh=_sc_mesh,
    scratch_types=[
        pltpu.VMEM((PPS,), I32),
        pltpu.VMEM((PPS, 2 * HID), F32),
        pltpu.SemaphoreType.DMA,
    ],
)


ROW_BLK = 2000
NB = N_NODES // ROW_BLK
NQ = N_REL // 2
ROW_BLK2 = 400
NB2 = N_NODES // ROW_BLK2


def _y_kernel(x_ref, wcat_ref, y_ref):
    y_ref[...] = jnp.dot(x_ref[...], wcat_ref[...],
                         preferred_element_type=F32)


def _y_matmul(x, wcat):
    return pl.pallas_call(
        _y_kernel,
        grid=(NQ, NB),
        in_specs=[
            pl.BlockSpec((ROW_BLK, HID), lambda q, i: (i, 0)),
            pl.BlockSpec((HID, 2 * HID), lambda q, i: (0, q)),
        ],
        out_specs=pl.BlockSpec((ROW_BLK, 2 * HID),
                               lambda q, i: (q * NB + i, 0)),
        out_shape=jax.ShapeDtypeStruct((NQ * N_NODES, 2 * HID), F32),
    )(x, wcat)


def _layer_kernel(relu, pad, x_ref, out2_ref, wroot_ref, b_ref, xn_ref):
    h = (jnp.dot(x_ref[...], wroot_ref[...], preferred_element_type=F32)
         + b_ref[...] + out2_ref[...])
    if relu:
        h = jnp.maximum(h, 0.0)
    if pad:
        h = jnp.concatenate([h, jnp.zeros_like(h)], axis=1)
    xn_ref[...] = h


def _layer_update(x, out2, wroot, b, relu, pad):
    n = x.shape[0]
    ow = 2 * HID if pad else HID
    return pl.pallas_call(
        functools.partial(_layer_kernel, relu, pad),
        grid=(NB2,),
        in_specs=[
            pl.BlockSpec((ROW_BLK2, HID), lambda i: (i, 0)),
            pl.BlockSpec((ROW_BLK2, HID), lambda i: (i, 0)),
            pl.BlockSpec((HID, HID), lambda i: (0, 0)),
            pl.BlockSpec((1, HID), lambda i: (0, 0)),
        ],
        out_specs=pl.BlockSpec((ROW_BLK2, ow), lambda i: (i, 0)),
        out_shape=jax.ShapeDtypeStruct((n, ow), F32),
    )(x, out2, wroot, b.reshape(1, HID))


def _head_kernel(h1_ref, h2_ref, w1a_ref, w1b_ref, b1_ref, w2_ref, b2_ref,
                 o_ref):
    h = (jnp.dot(h1_ref[...], w1a_ref[...], preferred_element_type=F32)
         + jnp.dot(h2_ref[...], w1b_ref[...], preferred_element_type=F32)
         + b1_ref[...])
    h = jnp.maximum(h, 0.0)
    o_ref[...] = jnp.dot(h, w2_ref[...], preferred_element_type=F32) + b2_ref[...]


def _head(h1, h2, wc1, bc1, wc2, bc2):
    zpad = jnp.zeros((HID, HID), F32)
    w1a = jnp.concatenate([wc1[:HID], zpad], axis=0)
    w1b = jnp.concatenate([wc1[HID:], zpad], axis=0)
    return pl.pallas_call(
        _head_kernel,
        out_shape=jax.ShapeDtypeStruct((BATCH, N_REL), F32),
    )(h1, h2, w1a, w1b, bc1.reshape(1, HID), wc2, bc2.reshape(1, N_REL))


def kernel(edge_index, edge_type, drug1_idx, drug2_idx, emb, Wr1, Wroot1, b1,
           Wr2, Wroot2, b2, Wc1, bc1, Wc2, bc2):
    src = edge_index[0].astype(I32)
    dst = edge_index[1].astype(I32)
    et = edge_type.astype(I32)
    d1 = drug1_idx.astype(I32)
    d2 = drug2_idx.astype(I32)

    cnt = _k1(dst, et)
    recip = _k1b(cnt)
    a_p, b_p, cnts = _k2(src, dst, et)

    Wcat1 = jnp.transpose(Wr1, (1, 0, 2)).reshape(HID, N_REL * HID)
    Wcat2 = jnp.transpose(Wr2, (1, 0, 2)).reshape(HID, N_REL * HID)

    y1 = _y_matmul(emb, Wcat1).reshape(2 * NQ * N_NODES, HID)
    out2_1 = _k3(y1, a_p, b_p, cnts, recip)[:N_NODES]
    x1 = _layer_update(emb, out2_1, Wroot1, b1, relu=True, pad=False)
    y2 = _y_matmul(x1, Wcat2).reshape(2 * NQ * N_NODES, HID)
    out2_2 = _k3(y2, a_p, b_p, cnts, recip)[:N_NODES]
    x2 = _layer_update(x1, out2_2, Wroot2, b2, relu=False, pad=True)

    h12 = _k4(x2, d1, d2)
    return _head(h12[0], h12[1], Wc1, bc1, Wc2, bc2)
